# Initial kernel scaffold; baseline (speedup 1.0000x reference)
import jax
import jax.numpy as jnp
from jax.experimental import pallas as pl


def kernel(write_weights, free_gate, read_weights, prev_usage):
    raise NotImplementedError("write your pallas kernel here")


# speedup vs baseline: 1.2573x; 1.2573x over previous
import jax
import jax.numpy as jnp
from jax.experimental import pallas as pl

B = 256
M = 16384
NW = 4
NR = 8

B_TILE = 8


def _body(ww_ref, fg_ref, rw_ref, prev_ref, out_ref):
    prev = prev_ref[...]
    alloc = 1.0 - (
        (1.0 - ww_ref[:, 0, :])
        * (1.0 - ww_ref[:, 1, :])
        * (1.0 - ww_ref[:, 2, :])
        * (1.0 - ww_ref[:, 3, :])
    )
    u = prev + (1.0 - prev) * alloc
    fg = fg_ref[...]
    free = fg[:, 0:1] * rw_ref[:, 0, :]
    for r in range(1, NR):
        free = free + fg[:, r : r + 1] * rw_ref[:, r, :]
    out_ref[...] = jnp.clip(u - free, 0.0, 1.0)


def kernel(write_weights, free_gate, read_weights, prev_usage):
    grid = (B // B_TILE,)
    return pl.pallas_call(
        _body,
        grid=grid,
        in_specs=[
            pl.BlockSpec((B_TILE, NW, M), lambda i: (i, 0, 0)),
            pl.BlockSpec((B_TILE, NR), lambda i: (i, 0)),
            pl.BlockSpec((B_TILE, NR, M), lambda i: (i, 0, 0)),
            pl.BlockSpec((B_TILE, M), lambda i: (i, 0)),
        ],
        out_specs=pl.BlockSpec((B_TILE, M), lambda i: (i, 0)),
        out_shape=jax.ShapeDtypeStruct((B, M), jnp.float32),
    )(write_weights, free_gate, read_weights, prev_usage)
